# baseline (speedup 1.0000x reference)
import jax
import jax.numpy as jnp
from jax import lax
from jax.experimental import pallas as pl
from jax.experimental.pallas import tpu as pltpu
from jax.experimental.pallas import tpu_sc as plsc

NUM_CORES = 2
NUM_SUBCORES = 16
NUM_WORKERS = NUM_CORES * NUM_SUBCORES
LANES = 16
EMBED_DIM = 64
PAD_DIM = 128
SEQ = 200
SENT_PER_CHUNK = 2
GROUP_OFFS = tuple(range(0, SEQ - LANES + 1, LANES)) + (SEQ - LANES,)


def _emb_body(idx_hbm, table_hbm, out_hbm, idx_v, rows0, rows1, g0, g1, s0, s1):
    n_sent = idx_hbm.shape[0] // NUM_WORKERS
    wid = lax.axis_index("s") * NUM_CORES + lax.axis_index("c")
    sent_base = wid * n_sent
    pltpu.sync_copy(idx_hbm.at[pl.ds(sent_base, n_sent), :], idx_v)

    rows = (rows0, rows1)
    gsem = (g0, g1)
    ssem = (s0, s1)
    n_chunks = n_sent // SENT_PER_CHUNK
    n_pairs = n_chunks // 2

    def start_gather(chunk_id, b):
        for s in range(SENT_PER_CHUNK):
            pltpu.async_copy(
                table_hbm.at[idx_v.at[chunk_id * SENT_PER_CHUNK + s]],
                rows[b].at[s],
                gsem[b],
            )

    def wait_gather(chunk_id, b):
        for s in range(SENT_PER_CHUNK):
            pltpu.make_async_copy(
                table_hbm.at[idx_v.at[chunk_id * SENT_PER_CHUNK + s]],
                rows[b].at[s],
                gsem[b],
            ).wait()

    def out_slice(chunk_id):
        off = pl.multiple_of(chunk_id * SENT_PER_CHUNK, SENT_PER_CHUNK)
        return out_hbm.at[pl.ds(sent_base + off, SENT_PER_CHUNK), :, :]

    def start_scatter(chunk_id, b):
        pltpu.async_copy(rows[b], out_slice(chunk_id), ssem[b])

    def wait_scatter(chunk_id, b):
        pltpu.make_async_copy(rows[b], out_slice(chunk_id), ssem[b]).wait()

    def scan_zeros(chunk_id):
        srow = pl.multiple_of(chunk_id * SENT_PER_CHUNK, SENT_PER_CHUNK)
        acc = jnp.zeros((LANES,), jnp.int32)
        for s in range(SENT_PER_CHUNK):
            for off in GROUP_OFFS:
                v = idx_v[srow + s, pl.ds(off, LANES)]
                acc = acc | (v == 0).astype(jnp.int32)
        return jnp.sum(acc)

    def patch(chunk_id, b):
        srow = pl.multiple_of(chunk_id * SENT_PER_CHUNK, SENT_PER_CHUNK)
        zv = jnp.zeros((2 * LANES,), jnp.bfloat16)
        for s in range(SENT_PER_CHUNK):
            for off in GROUP_OFFS:
                v = idx_v[srow + s, pl.ds(off, LANES)]
                nzg = jnp.sum((v == 0).astype(jnp.int32))

                @pl.when(nzg > 0)
                def _(s=s, off=off, v=v):
                    for l in range(LANES):
                        zval = v[l]

                        @pl.when(zval == 0)
                        def _(s=s, off=off, l=l):
                            for q in range(EMBED_DIM // (2 * LANES)):
                                rows[b][s, off + l, pl.ds(q * 2 * LANES, 2 * LANES)] = zv

    def pair(p, c):
        a = 2 * p
        bc = 2 * p + 1
        nza = scan_zeros(a)
        wait_gather(a, 0)

        @pl.when(nza > 0)
        def _():
            patch(a, 0)

        start_scatter(a, 0)

        @pl.when(p > 0)
        def _():
            wait_scatter(bc - 2, 1)

        start_gather(bc, 1)
        nzb = scan_zeros(bc)
        wait_gather(bc, 1)

        @pl.when(nzb > 0)
        def _():
            patch(bc, 1)

        start_scatter(bc, 1)
        wait_scatter(a, 0)

        @pl.when(p < n_pairs - 1)
        def _():
            start_gather(a + 2, 0)

        return c

    start_gather(0, 0)
    lax.fori_loop(0, n_pairs, pair, 0)
    wait_scatter(n_chunks - 1, 1)


def kernel(input, table):
    n_sentences, seq = input.shape
    idx = input.astype(jnp.int32)
    tbl = jnp.pad(table.astype(jnp.bfloat16), ((0, 0), (0, PAD_DIM - EMBED_DIM)))
    mesh = plsc.VectorSubcoreMesh(
        core_axis_name="c",
        subcore_axis_name="s",
        num_cores=NUM_CORES,
        num_subcores=NUM_SUBCORES,
    )
    n_per_w = n_sentences // NUM_WORKERS
    out = pl.kernel(
        _emb_body,
        out_type=jax.ShapeDtypeStruct((n_sentences, seq, PAD_DIM), jnp.bfloat16),
        mesh=mesh,
        compiler_params=pltpu.CompilerParams(
            use_tc_tiling_on_sc=False, needs_layout_passes=False
        ),
        scratch_types=[
            pltpu.VMEM((n_per_w, seq), jnp.int32),
            pltpu.VMEM((SENT_PER_CHUNK, seq, PAD_DIM), jnp.bfloat16),
            pltpu.VMEM((SENT_PER_CHUNK, seq, PAD_DIM), jnp.bfloat16),
            pltpu.SemaphoreType.DMA,
            pltpu.SemaphoreType.DMA,
            pltpu.SemaphoreType.DMA,
            pltpu.SemaphoreType.DMA,
        ],
    )(idx, tbl)
    return out[:, :, :EMBED_DIM].astype(jnp.float32)


# speedup vs baseline: 2.5867x; 2.5867x over previous
import jax
import jax.numpy as jnp
from jax import lax
from jax.experimental import pallas as pl
from jax.experimental.pallas import tpu as pltpu
from jax.experimental.pallas import tpu_sc as plsc

NUM_CORES = 2
NUM_SUBCORES = 16
NUM_WORKERS = NUM_CORES * NUM_SUBCORES
LANES = 16
EMBED_DIM = 64
PAD_DIM = 128
SEQ = 200
SENT_PER_CHUNK = 2
GROUP_OFFS = tuple(range(0, SEQ - LANES + 1, LANES)) + (SEQ - LANES,)


def _emb_body(idx_hbm, table_hbm, out_hbm, idx_v, rows0, rows1, g0, g1, s0, s1):
    n_sent = idx_hbm.shape[0] // NUM_WORKERS
    wid = lax.axis_index("s") * NUM_CORES + lax.axis_index("c")
    sent_base = wid * n_sent
    pltpu.sync_copy(idx_hbm.at[pl.ds(sent_base, n_sent), :], idx_v)

    rows = (rows0, rows1)
    gsem = (g0, g1)
    ssem = (s0, s1)
    n_chunks = n_sent // SENT_PER_CHUNK
    n_pairs = n_chunks // 2

    def start_gather(chunk_id, b):
        for s in range(SENT_PER_CHUNK):
            pltpu.async_copy(
                table_hbm.at[idx_v.at[chunk_id * SENT_PER_CHUNK + s]],
                rows[b].at[s],
                gsem[b],
            )

    def wait_gather(chunk_id, b):
        for s in range(SENT_PER_CHUNK):
            pltpu.make_async_copy(
                table_hbm.at[idx_v.at[chunk_id * SENT_PER_CHUNK + s]],
                rows[b].at[s],
                gsem[b],
            ).wait()

    def out_slice(chunk_id):
        off = pl.multiple_of(chunk_id * SENT_PER_CHUNK, SENT_PER_CHUNK)
        return out_hbm.at[pl.ds(sent_base + off, SENT_PER_CHUNK), :, :]

    def start_scatter(chunk_id, b):
        pltpu.async_copy(rows[b], out_slice(chunk_id), ssem[b])

    def wait_scatter(chunk_id, b):
        pltpu.make_async_copy(rows[b], out_slice(chunk_id), ssem[b]).wait()

    def scan_zeros(chunk_id):
        srow = pl.multiple_of(chunk_id * SENT_PER_CHUNK, SENT_PER_CHUNK)
        acc = jnp.zeros((LANES,), jnp.int32)
        for s in range(SENT_PER_CHUNK):
            for off in GROUP_OFFS:
                v = idx_v[srow + s, pl.ds(off, LANES)]
                acc = acc | (v == 0).astype(jnp.int32)
        return jnp.sum(acc)

    def patch(chunk_id, b):
        srow = pl.multiple_of(chunk_id * SENT_PER_CHUNK, SENT_PER_CHUNK)
        z = jnp.zeros((LANES,), jnp.float32)
        for s in range(SENT_PER_CHUNK):
            sv = jnp.full((LANES,), s, jnp.int32)
            for off in GROUP_OFFS:
                v = idx_v[srow + s, pl.ds(off, LANES)]
                m = v == 0
                nzg = jnp.sum(m.astype(jnp.int32))

                @pl.when(nzg > 0)
                def _(s=s, sv=sv, off=off, m=m):
                    tokv = lax.iota(jnp.int32, LANES) + off
                    for k in range(EMBED_DIM):
                        plsc.store_scatter(
                            rows[b],
                            [sv, tokv, jnp.full((LANES,), k, jnp.int32)],
                            z,
                            mask=m,
                        )

    def pair(p, c):
        a = 2 * p
        bc = 2 * p + 1
        nza = scan_zeros(a)
        wait_gather(a, 0)

        @pl.when(nza > 0)
        def _():
            patch(a, 0)

        start_scatter(a, 0)

        @pl.when(p > 0)
        def _():
            wait_scatter(bc - 2, 1)

        start_gather(bc, 1)
        nzb = scan_zeros(bc)
        wait_gather(bc, 1)

        @pl.when(nzb > 0)
        def _():
            patch(bc, 1)

        start_scatter(bc, 1)
        wait_scatter(a, 0)

        @pl.when(p < n_pairs - 1)
        def _():
            start_gather(a + 2, 0)

        return c

    start_gather(0, 0)
    lax.fori_loop(0, n_pairs, pair, 0)
    wait_scatter(n_chunks - 1, 1)


def kernel(input, table):
    n_sentences, seq = input.shape
    idx = input.astype(jnp.int32)
    tbl = jnp.pad(table, ((0, 0), (0, PAD_DIM - EMBED_DIM)))
    mesh = plsc.VectorSubcoreMesh(
        core_axis_name="c",
        subcore_axis_name="s",
        num_cores=NUM_CORES,
        num_subcores=NUM_SUBCORES,
    )
    n_per_w = n_sentences // NUM_WORKERS
    out = pl.kernel(
        _emb_body,
        out_type=jax.ShapeDtypeStruct((n_sentences, seq, PAD_DIM), jnp.float32),
        mesh=mesh,
        compiler_params=pltpu.CompilerParams(
            use_tc_tiling_on_sc=False, needs_layout_passes=False
        ),
        scratch_types=[
            pltpu.VMEM((n_per_w, seq), jnp.int32),
            pltpu.VMEM((SENT_PER_CHUNK, seq, PAD_DIM), jnp.float32),
            pltpu.VMEM((SENT_PER_CHUNK, seq, PAD_DIM), jnp.float32),
            pltpu.SemaphoreType.DMA,
            pltpu.SemaphoreType.DMA,
            pltpu.SemaphoreType.DMA,
            pltpu.SemaphoreType.DMA,
        ],
    )(idx, tbl)
    return out[:, :, :EMBED_DIM]


# speedup vs baseline: 2.8052x; 1.0845x over previous
import jax
import jax.numpy as jnp
from jax import lax
from jax.experimental import pallas as pl
from jax.experimental.pallas import tpu as pltpu
from jax.experimental.pallas import tpu_sc as plsc

NUM_CORES = 2
NUM_SUBCORES = 16
NUM_WORKERS = NUM_CORES * NUM_SUBCORES
LANES = 16
EMBED_DIM = 64
PAD_DIM = 128
SEQ = 200
SENT_PER_CHUNK = 2
GROUP_OFFS = tuple(range(0, SEQ - LANES + 1, LANES)) + (SEQ - LANES,)


def _emb_body(idx_hbm, table_hbm, out_hbm, idx_v, rows0, rows1, g0, g1, s0, s1):
    n_sent = idx_hbm.shape[0] // NUM_WORKERS
    wid = lax.axis_index("s") * NUM_CORES + lax.axis_index("c")
    sent_base = wid * n_sent
    pltpu.sync_copy(idx_hbm.at[pl.ds(sent_base, n_sent), :], idx_v)

    rows = (rows0, rows1)
    gsem = (g0, g1)
    ssem = (s0, s1)
    n_chunks = n_sent // SENT_PER_CHUNK
    n_pairs = n_chunks // 2

    def start_gather(chunk_id, b):
        for s in range(SENT_PER_CHUNK):
            pltpu.async_copy(
                table_hbm.at[idx_v.at[chunk_id * SENT_PER_CHUNK + s]],
                rows[b].at[s],
                gsem[b],
            )

    def wait_gather(chunk_id, b):
        for s in range(SENT_PER_CHUNK):
            pltpu.make_async_copy(
                table_hbm.at[idx_v.at[chunk_id * SENT_PER_CHUNK + s]],
                rows[b].at[s],
                gsem[b],
            ).wait()

    def out_slice(chunk_id):
        off = pl.multiple_of(chunk_id * SENT_PER_CHUNK, SENT_PER_CHUNK)
        return out_hbm.at[pl.ds(sent_base + off, SENT_PER_CHUNK), :, pl.ds(0, EMBED_DIM)]

    def start_scatter(chunk_id, b):
        pltpu.async_copy(rows[b], out_slice(chunk_id), ssem[b])

    def wait_scatter(chunk_id, b):
        pltpu.make_async_copy(rows[b], out_slice(chunk_id), ssem[b]).wait()

    def scan_zeros(chunk_id):
        srow = pl.multiple_of(chunk_id * SENT_PER_CHUNK, SENT_PER_CHUNK)
        acc = jnp.zeros((LANES,), jnp.int32)
        for s in range(SENT_PER_CHUNK):
            for off in GROUP_OFFS:
                v = idx_v[srow + s, pl.ds(off, LANES)]
                acc = acc | (v == 0).astype(jnp.int32)
        return jnp.sum(acc)

    def patch(chunk_id, b):
        srow = pl.multiple_of(chunk_id * SENT_PER_CHUNK, SENT_PER_CHUNK)
        z = jnp.zeros((LANES,), jnp.float32)
        for s in range(SENT_PER_CHUNK):
            sv = jnp.full((LANES,), s, jnp.int32)
            for off in GROUP_OFFS:
                v = idx_v[srow + s, pl.ds(off, LANES)]
                m = v == 0
                nzg = jnp.sum(m.astype(jnp.int32))

                @pl.when(nzg > 0)
                def _(s=s, sv=sv, off=off, m=m):
                    tokv = lax.iota(jnp.int32, LANES) + off
                    for k in range(EMBED_DIM):
                        plsc.store_scatter(
                            rows[b],
                            [sv, tokv, jnp.full((LANES,), k, jnp.int32)],
                            z,
                            mask=m,
                        )

    def pair(p, c):
        a = 2 * p
        bc = 2 * p + 1
        nza = scan_zeros(a)
        wait_gather(a, 0)

        @pl.when(nza > 0)
        def _():
            patch(a, 0)

        start_scatter(a, 0)

        @pl.when(p > 0)
        def _():
            wait_scatter(bc - 2, 1)

        start_gather(bc, 1)
        nzb = scan_zeros(bc)
        wait_gather(bc, 1)

        @pl.when(nzb > 0)
        def _():
            patch(bc, 1)

        start_scatter(bc, 1)
        wait_scatter(a, 0)

        @pl.when(p < n_pairs - 1)
        def _():
            start_gather(a + 2, 0)

        return c

    start_gather(0, 0)
    lax.fori_loop(0, n_pairs, pair, 0)
    wait_scatter(n_chunks - 1, 1)


def kernel(input, table):
    n_sentences, seq = input.shape
    idx = input.astype(jnp.int32)
    mesh = plsc.VectorSubcoreMesh(
        core_axis_name="c",
        subcore_axis_name="s",
        num_cores=NUM_CORES,
        num_subcores=NUM_SUBCORES,
    )
    n_per_w = n_sentences // NUM_WORKERS
    out = pl.kernel(
        _emb_body,
        out_type=jax.ShapeDtypeStruct((n_sentences, seq, PAD_DIM), jnp.float32),
        mesh=mesh,
        compiler_params=pltpu.CompilerParams(
            use_tc_tiling_on_sc=False, needs_layout_passes=False
        ),
        scratch_types=[
            pltpu.VMEM((n_per_w, seq), jnp.int32),
            pltpu.VMEM((SENT_PER_CHUNK, seq, EMBED_DIM), jnp.float32),
            pltpu.VMEM((SENT_PER_CHUNK, seq, EMBED_DIM), jnp.float32),
            pltpu.SemaphoreType.DMA,
            pltpu.SemaphoreType.DMA,
            pltpu.SemaphoreType.DMA,
            pltpu.SemaphoreType.DMA,
        ],
    )(idx, table)
    return out[:, :, :EMBED_DIM]


# speedup vs baseline: 3.0133x; 1.0742x over previous
import jax
import jax.numpy as jnp
from jax import lax
from jax.experimental import pallas as pl
from jax.experimental.pallas import tpu as pltpu
from jax.experimental.pallas import tpu_sc as plsc

NUM_CORES = 2
NUM_SUBCORES = 16
NUM_WORKERS = NUM_CORES * NUM_SUBCORES
LANES = 16
EMBED_DIM = 64
PAD_DIM = 128
SEQ = 200
SENT_PER_CHUNK = 2
GROUP_OFFS = tuple(range(0, SEQ - LANES + 1, LANES)) + (SEQ - LANES,)


def _emb_body(idx_hbm, table_hbm, out_hbm, idx_v, rows0, rows1, g0, g1, s0, s1):
    n_sent = idx_hbm.shape[0] // NUM_WORKERS
    wid = lax.axis_index("s") * NUM_CORES + lax.axis_index("c")
    sent_base = wid * n_sent
    pltpu.sync_copy(idx_hbm.at[pl.ds(sent_base, n_sent), :], idx_v)

    tail_keep = lax.iota(jnp.int32, LANES) < (GROUP_OFFS[-2] + LANES - GROUP_OFFS[-1])

    def dbl(j, c):
        for off in GROUP_OFFS[:-1]:
            idx_v[j, pl.ds(off, LANES)] = idx_v[j, pl.ds(off, LANES)] * 2
        off = GROUP_OFFS[-1]
        v = idx_v[j, pl.ds(off, LANES)]
        idx_v[j, pl.ds(off, LANES)] = jnp.where(tail_keep, v, v * 2)
        return c

    lax.fori_loop(0, n_sent, dbl, 0)

    rows = (rows0, rows1)
    gsem = (g0, g1)
    ssem = (s0, s1)
    n_chunks = n_sent // SENT_PER_CHUNK
    n_pairs = n_chunks // 2

    def start_gather(chunk_id, b):
        for s in range(SENT_PER_CHUNK):
            pltpu.async_copy(
                table_hbm.at[idx_v.at[chunk_id * SENT_PER_CHUNK + s]],
                rows[b].at[s],
                gsem[b],
            )

    def wait_gather(chunk_id, b):
        for s in range(SENT_PER_CHUNK):
            pltpu.make_async_copy(
                table_hbm.at[idx_v.at[chunk_id * SENT_PER_CHUNK + s]],
                rows[b].at[s],
                gsem[b],
            ).wait()

    def out_slice(chunk_id):
        off = pl.multiple_of(chunk_id * SENT_PER_CHUNK, SENT_PER_CHUNK)
        return out_hbm.at[pl.ds(sent_base + off, SENT_PER_CHUNK), :, pl.ds(0, EMBED_DIM)]

    def start_scatter(chunk_id, b):
        pltpu.async_copy(rows[b], out_slice(chunk_id), ssem[b])

    def wait_scatter(chunk_id, b):
        pltpu.make_async_copy(rows[b], out_slice(chunk_id), ssem[b]).wait()

    def scan_zeros(chunk_id):
        srow = pl.multiple_of(chunk_id * SENT_PER_CHUNK, SENT_PER_CHUNK)
        acc = jnp.zeros((LANES,), jnp.int32)
        for s in range(SENT_PER_CHUNK):
            for off in GROUP_OFFS:
                v = idx_v[srow + s, pl.ds(off, LANES)]
                acc = acc | (v == 0).astype(jnp.int32)
        return jnp.sum(acc)

    def patch(chunk_id, b):
        srow = pl.multiple_of(chunk_id * SENT_PER_CHUNK, SENT_PER_CHUNK)
        z = jnp.zeros((LANES,), jnp.float32)
        for s in range(SENT_PER_CHUNK):
            sv = jnp.full((LANES,), s, jnp.int32)
            for off in GROUP_OFFS:
                v = idx_v[srow + s, pl.ds(off, LANES)]
                m = v == 0
                nzg = jnp.sum(m.astype(jnp.int32))

                @pl.when(nzg > 0)
                def _(s=s, sv=sv, off=off, m=m):
                    tokv = lax.iota(jnp.int32, LANES) + off
                    for k in range(EMBED_DIM):
                        plsc.store_scatter(
                            rows[b],
                            [sv, tokv, jnp.full((LANES,), k, jnp.int32)],
                            z,
                            mask=m,
                        )

    def pair(p, c):
        a = 2 * p
        bc = 2 * p + 1
        nza = scan_zeros(a)
        wait_gather(a, 0)

        @pl.when(nza > 0)
        def _():
            patch(a, 0)

        start_scatter(a, 0)

        @pl.when(p > 0)
        def _():
            wait_scatter(bc - 2, 1)

        start_gather(bc, 1)
        nzb = scan_zeros(bc)
        wait_gather(bc, 1)

        @pl.when(nzb > 0)
        def _():
            patch(bc, 1)

        start_scatter(bc, 1)
        wait_scatter(a, 0)

        @pl.when(p < n_pairs - 1)
        def _():
            start_gather(a + 2, 0)

        return c

    start_gather(0, 0)
    lax.fori_loop(0, n_pairs, pair, 0)
    wait_scatter(n_chunks - 1, 1)


def kernel(input, table):
    n_sentences, seq = input.shape
    idx = input.astype(jnp.int32)
    mesh = plsc.VectorSubcoreMesh(
        core_axis_name="c",
        subcore_axis_name="s",
        num_cores=NUM_CORES,
        num_subcores=NUM_SUBCORES,
    )
    n_per_w = n_sentences // NUM_WORKERS
    out = pl.kernel(
        _emb_body,
        out_type=jax.ShapeDtypeStruct((n_sentences, seq, PAD_DIM), jnp.float32),
        mesh=mesh,
        compiler_params=pltpu.CompilerParams(
            use_tc_tiling_on_sc=False, needs_layout_passes=False
        ),
        scratch_types=[
            pltpu.VMEM((n_per_w, seq), jnp.int32),
            pltpu.VMEM((SENT_PER_CHUNK, seq, EMBED_DIM), jnp.float32),
            pltpu.VMEM((SENT_PER_CHUNK, seq, EMBED_DIM), jnp.float32),
            pltpu.SemaphoreType.DMA,
            pltpu.SemaphoreType.DMA,
            pltpu.SemaphoreType.DMA,
            pltpu.SemaphoreType.DMA,
        ],
    )(idx, jnp.pad(table, ((0, 0), (0, PAD_DIM - EMBED_DIM))).reshape(2 * table.shape[0], EMBED_DIM))
    return out[:, :, :EMBED_DIM]
